# baseline (speedup 1.0000x reference)
import functools

import jax
import jax.numpy as jnp
from jax import lax
from jax.experimental import pallas as pl
from jax.experimental.pallas import tpu as pltpu
from jax.experimental.pallas import tpu_sc as plsc

F32 = jnp.float32
EPS = 1e-4
BLK = 512
GCH = 128
GROUND = 1024


def _ru(a, m):
    return (a + m - 1) // m * m


@functools.lru_cache(None)
def _make_gather(T, C, NTAB):
    NC, NS = 2, 16
    NW = NC * NS
    assert T % (NW * GCH) == 0
    per_w = T // NW
    rounds = per_w // GROUND
    rem = (per_w % GROUND) // GCH
    mesh = plsc.VectorSubcoreMesh(core_axis_name="c", subcore_axis_name="s")

    def body(table_hbm, idx_hbm, out_hbm, idx_v, rows_v, sem):
        wid = lax.axis_index("s") * NC + lax.axis_index("c")
        wbase = wid * per_w

        def do_round(base, nch):
            n = nch * GCH
            pltpu.sync_copy(idx_hbm.at[pl.ds(base, n)], idx_v.at[pl.ds(0, n)])
            descs = [
                pltpu.async_copy(
                    table_hbm.at[idx_v.at[pl.ds(b * GCH, GCH)]],
                    rows_v.at[pl.ds(b * GCH, GCH)],
                    sem,
                )
                for b in range(nch)
            ]
            for d in descs:
                d.wait()
            pltpu.sync_copy(rows_v.at[pl.ds(0, n)], out_hbm.at[pl.ds(base, n)])

        if rounds:
            def loop_body(r, carry):
                do_round(wbase + r * GROUND, GROUND // GCH)
                return carry
            lax.fori_loop(0, rounds, loop_body, 0)
        if rem:
            do_round(wbase + rounds * GROUND, rem)

    return pl.kernel(
        body,
        out_type=jax.ShapeDtypeStruct((T, C), F32),
        mesh=mesh,
        compiler_params=pltpu.CompilerParams(use_tc_tiling_on_sc=False),
        scratch_types=[
            pltpu.VMEM((GROUND,), jnp.int32),
            pltpu.VMEM((GROUND, C), F32),
            pltpu.SemaphoreType.DMA,
        ],
    )


@functools.lru_cache(None)
def _make_matmulK(R, K, C, D):
    rb = R // BLK

    def body(g_ref, w_ref, o_ref):
        @pl.when(pl.program_id(1) == 0)
        def _():
            o_ref[...] = jnp.zeros_like(o_ref)
        o_ref[...] += jnp.dot(g_ref[...], w_ref[0],
                              preferred_element_type=F32)

    return pl.pallas_call(
        body,
        grid=(rb, K),
        in_specs=[
            pl.BlockSpec((BLK, C), lambda i, k: (k * rb + i, 0)),
            pl.BlockSpec((1, C, D), lambda i, k: (k, 0, 0)),
        ],
        out_specs=pl.BlockSpec((BLK, D), lambda i, k: (i, 0)),
        out_shape=jax.ShapeDtypeStruct((R, D), F32),
    )


@functools.lru_cache(None)
def _make_upmm(R, C, D):
    def body(g_ref, m_ref, w_ref, o_ref):
        @pl.when(pl.program_id(1) == 0)
        def _():
            o_ref[...] = jnp.zeros_like(o_ref)
        x = g_ref[...] * m_ref[0]
        o_ref[...] += jnp.dot(x, w_ref[0], preferred_element_type=F32)

    return pl.pallas_call(
        body,
        grid=(R // BLK, 8),
        in_specs=[
            pl.BlockSpec((BLK, C), lambda i, o: (i, 0)),
            pl.BlockSpec((1, BLK, 1), lambda i, o: (o, i, 0)),
            pl.BlockSpec((1, C, D), lambda i, o: (o, 0, 0)),
        ],
        out_specs=pl.BlockSpec((BLK, D), lambda i, o: (i, 0)),
        out_shape=jax.ShapeDtypeStruct((R, D), F32),
    )


@functools.lru_cache(None)
def _make_stats(R, C):
    def body(x_ref, o_ref):
        @pl.when(pl.program_id(0) == 0)
        def _():
            o_ref[...] = jnp.zeros_like(o_ref)
        x = x_ref[...]
        o_ref[0:1, :] += jnp.sum(x, 0, keepdims=True)
        o_ref[1:2, :] += jnp.sum(x * x, 0, keepdims=True)

    return pl.pallas_call(
        body,
        grid=(R // BLK,),
        in_specs=[pl.BlockSpec((BLK, C), lambda i: (i, 0))],
        out_specs=pl.BlockSpec((2, C), lambda i: (0, 0)),
        out_shape=jax.ShapeDtypeStruct((2, C), F32),
    )


@functools.lru_cache(None)
def _make_norm(Rin, C, Rout, Nreal):
    inv_n = 1.0 / Nreal

    def body(x_ref, s_ref, g_ref, b_ref, o_ref):
        i = pl.program_id(0)
        mu = s_ref[0:1, :] * inv_n
        var = s_ref[1:2, :] * inv_n - mu * mu
        rs = lax.rsqrt(var + EPS)
        y = (x_ref[...] - mu) * rs * g_ref[...] + b_ref[...]
        y = jnp.maximum(y, 0.0)
        row = i * BLK + lax.broadcasted_iota(jnp.int32, (BLK, C), 0)
        o_ref[...] = jnp.where(row < Nreal, y, 0.0)

    return pl.pallas_call(
        body,
        grid=(Rin // BLK,),
        in_specs=[
            pl.BlockSpec((BLK, C), lambda i: (i, 0)),
            pl.BlockSpec((2, C), lambda i: (0, 0)),
            pl.BlockSpec((1, C), lambda i: (0, 0)),
            pl.BlockSpec((1, C), lambda i: (0, 0)),
        ],
        out_specs=pl.BlockSpec((BLK, C), lambda i: (i, 0)),
        out_shape=jax.ShapeDtypeStruct((Rout, C), F32),
    )


def _bnrelu(x, g, b, nreal, rout=None):
    R, C = x.shape
    st = _make_stats(R, C)(x)
    return _make_norm(R, C, rout if rout is not None else R, nreal)(
        x, st, g.reshape(1, C), b.reshape(1, C))


def kernel(feats, params, sm_nbr0, sm_nbr1, sm_nbr2, sm_nbr3,
           parent0, parent1, parent2, off0, off1, off2):
    sms = [sm_nbr0, sm_nbr1, sm_nbr2, sm_nbr3]
    parents = [parent0, parent1, parent2]
    offs = [off0, off1, off2]
    Ns = [s.shape[0] for s in sms]
    Nr = [_ru(n + 1, BLK) for n in Ns]
    CIN = feats.shape[1]
    NP = [params[f"W_blk{i}"].shape[1] for i in range(3)] \
        + [params["W_mid"].shape[1]]
    M = params["W_sc1"].shape[2]

    def prep27(i):
        T = _ru(27 * Nr[i], 4096)
        nb = jnp.pad(sms[i].T.astype(jnp.int32),
                     ((0, 0), (0, Nr[i] - Ns[i])),
                     constant_values=Ns[i]).reshape(-1)
        return jnp.pad(nb, (0, T - 27 * Nr[i]), constant_values=Ns[i]), T

    idx27 = [prep27(i) for i in range(4)]

    def prep_ch(i):
        T = 8 * Nr[i + 1]
        pos = offs[i].astype(jnp.int32) * Nr[i + 1] + parents[i].astype(jnp.int32)
        ch = jnp.full((T,), Ns[i], jnp.int32)
        return ch.at[pos].set(jnp.arange(Ns[i], dtype=jnp.int32)), T

    idxch = [prep_ch(i) for i in range(3)]

    def prep_par(i):
        T = _ru(Nr[i], 4096)
        p = jnp.pad(parents[i].astype(jnp.int32), (0, T - Ns[i]),
                    constant_values=Ns[i + 1])
        return p, T

    idxpar = [prep_par(i) for i in range(3)]
    masks = [
        jnp.pad(jax.nn.one_hot(offs[i], 8, dtype=F32).T,
                ((0, 0), (0, Nr[i] - Ns[i]))).reshape(8, Nr[i], 1)
        for i in range(3)
    ]

    def subm(table, lvl, W):
        idx, T = idx27[lvl]
        C = table.shape[1]
        G = _make_gather(T, C, table.shape[0])(table, idx)
        return _make_matmulK(Nr[lvl], 27, C, W.shape[2])(G, W)

    cpad = _ru(CIN, 16) - CIN
    f16 = jnp.pad(feats, ((0, Nr[0] - Ns[0]), (0, cpad)))
    w1 = jnp.pad(params["W_sc1"], ((0, 0), (0, cpad), (0, 0)))
    x = subm(f16, 0, w1)

    inter = []
    for i in range(3):
        xn = _bnrelu(x, params[f"g_blk{i}"], params[f"b_blk{i}"], Ns[i])
        x = subm(xn, i, params[f"W_blk{i}"])
        inter.append(x)
        xn = _bnrelu(x, params[f"g_dn{i}"], params[f"b_dn{i}"], Ns[i])
        idx, T = idxch[i]
        G = _make_gather(T, NP[i], Nr[i])(xn, idx)
        x = _make_matmulK(Nr[i + 1], 8, NP[i], NP[i + 1])(G, params[f"W_dn{i}"])

    xn = _bnrelu(x, params["g_mid"], params["b_mid"], Ns[3])
    x = subm(xn, 3, params["W_mid"])

    for i in (2, 1, 0):
        yn = _bnrelu(x, params[f"g_up{i}"], params[f"b_up{i}"], Ns[i + 1])
        idx, T = idxpar[i]
        Gp = _make_gather(T, NP[i + 1], Nr[i + 1])(yn, idx)
        xup = _make_upmm(Nr[i], NP[i + 1], NP[i])(Gp, masks[i],
                                                  params[f"W_up{i}"])
        xc = jnp.concatenate([inter[i], xup], axis=1)
        xn = _bnrelu(xc, params[f"g_aj{i}"], params[f"b_aj{i}"], Ns[i])
        x = subm(xn, i, params[f"W_aj{i}"])

    return _bnrelu(x, params["g_out"], params["b_out"], Ns[0], rout=Ns[0])
